# baseline (speedup 1.0000x reference)
import functools

import jax
import jax.numpy as jnp
from jax import lax
from jax.experimental import pallas as pl
from jax.experimental.pallas import tpu as pltpu
from jax.experimental.pallas import tpu_sc as plsc

_N_NODES = 10000
_N_PAD = 10240
_N_EDGES = 320000
_DIM = 128
_NC = 2
_NS = 16
_NW = _NC * _NS
_K = 80
_CPW = 128
_E_PAD = _NW * _CPW * _K
_RPT = _N_PAD // _NS
_ZR = _K
_NB = 4


def _sc_aggregate(x, src_p, dst_p):
    mesh = plsc.VectorSubcoreMesh(core_axis_name="c", subcore_axis_name="s")

    @functools.partial(
        pl.kernel,
        out_type=jax.ShapeDtypeStruct((_NC, _N_PAD, _DIM), jnp.float32),
        mesh=mesh,
        scratch_types=[
            pltpu.VMEM_SHARED((_N_PAD, _DIM), jnp.float32),
            [pltpu.VMEM((_K, _DIM), jnp.float32)] * _NB,
            [pltpu.VMEM((_K,), jnp.int32)] * _NB,
            pltpu.VMEM((_K,), jnp.int32),
            [pltpu.SemaphoreType.DMA] * _NB,
        ],
    )
    def k(x_hbm, src_hbm, dst_hbm, out_hbm, accum, ring, sidx, dbuf, gsem):
        c = lax.axis_index("c")
        s = lax.axis_index("s")
        w = s * _NC + c
        e0 = w * _CPW * _K

        def zero_row(r, carry):
            for j in range(_DIM // 16):
                ring[0][r, pl.ds(j * 16, 16)] = jnp.zeros((16,), jnp.float32)
            return carry

        lax.fori_loop(0, _ZR, zero_row, 0)
        row0 = s * _RPT
        for j in range(_RPT // _ZR):
            pltpu.sync_copy(ring[0], accum.at[pl.ds(row0 + j * _ZR, _ZR)])
        plsc.subcore_barrier()

        def batch(i, carry):
            j0 = _NB * i
            for b in range(_NB):
                pltpu.sync_copy(
                    src_hbm.at[pl.ds(e0 + (j0 + b) * _K, _K)], sidx[b])
            descs = [
                pltpu.async_copy(x_hbm.at[sidx[b]], ring[b], gsem[b])
                for b in range(_NB)
            ]
            for b in range(_NB):
                pltpu.sync_copy(
                    dst_hbm.at[pl.ds(e0 + (j0 + b) * _K, _K)], dbuf)
                descs[b].wait()
                pltpu.sync_copy(ring[b], accum.at[dbuf], add=True)
            return carry

        lax.fori_loop(0, _CPW // _NB, batch, 0)
        plsc.subcore_barrier()

        for j in range(_RPT // _ZR):
            r = row0 + j * _ZR
            pltpu.sync_copy(accum.at[pl.ds(r, _ZR)], ring[0])
            pltpu.sync_copy(ring[0], out_hbm.at[c].at[pl.ds(r, _ZR)])

    return k(x, src_p, dst_p)


def _mm_relu(partials, W):
    blk = 1024

    def body(p0_ref, p1_ref, w_ref, o_ref):
        ssum = p0_ref[...] + p1_ref[...]
        o_ref[...] = jnp.maximum(
            jnp.dot(ssum, w_ref[...], preferred_element_type=jnp.float32),
            0.0)

    return pl.pallas_call(
        body,
        grid=(_N_PAD // blk,),
        in_specs=[
            pl.BlockSpec((blk, _DIM), lambda i: (i, 0)),
            pl.BlockSpec((blk, _DIM), lambda i: (i, 0)),
            pl.BlockSpec((_DIM, _DIM), lambda i: (0, 0)),
        ],
        out_specs=pl.BlockSpec((blk, _DIM), lambda i: (i, 0)),
        out_shape=jax.ShapeDtypeStruct((_N_NODES, _DIM), jnp.float32),
    )(partials[0], partials[1], W)


def kernel(x, edge_index, W):
    src = edge_index[1].astype(jnp.int32)
    dst = edge_index[0].astype(jnp.int32)
    npad = _E_PAD - _N_EDGES
    src_p = jnp.concatenate([src, jnp.zeros((npad,), jnp.int32)])
    dst_p = jnp.concatenate([dst, jnp.full((npad,), _N_NODES, jnp.int32)])
    partials = _sc_aggregate(x, src_p, dst_p)
    return _mm_relu(partials, W)


# speedup vs baseline: 1.8990x; 1.8990x over previous
import functools

import jax
import jax.numpy as jnp
from jax import lax
from jax.experimental import pallas as pl
from jax.experimental.pallas import tpu as pltpu
from jax.experimental.pallas import tpu_sc as plsc

_N_NODES = 10000
_N_PAD = 10240
_N_EDGES = 320000
_DIM = 128
_NC = 2
_NS = 16
_NW = _NC * _NS
_K = 80
_CPW = 125
_RPT = _N_PAD // _NS
_ZR = _K


def _sc_aggregate(x, src_p, dst_p):
    mesh = plsc.VectorSubcoreMesh(core_axis_name="c", subcore_axis_name="s")

    @functools.partial(
        pl.kernel,
        out_type=jax.ShapeDtypeStruct((_NC, _N_PAD, _DIM), jnp.float32),
        mesh=mesh,
        scratch_types=[
            pltpu.VMEM_SHARED((_N_PAD, _DIM), jnp.float32),
            pltpu.VMEM((_K, _DIM), jnp.float32),
            pltpu.VMEM((_K,), jnp.int32),
            pltpu.VMEM((_K,), jnp.int32),
            pltpu.SemaphoreType.DMA,
        ],
    )
    def k(x_hbm, src_hbm, dst_hbm, out_hbm, accum, rows, sbuf, dbuf, gsem):
        c = lax.axis_index("c")
        s = lax.axis_index("s")
        w = s * _NC + c
        e0 = w * _CPW * _K

        def zero_row(r, carry):
            for j in range(_DIM // 16):
                rows[r, pl.ds(j * 16, 16)] = jnp.zeros((16,), jnp.float32)
            return carry

        lax.fori_loop(0, _ZR, zero_row, 0)
        row0 = s * _RPT
        for j in range(_RPT // _ZR):
            pltpu.sync_copy(rows, accum.at[pl.ds(row0 + j * _ZR, _ZR)])
        plsc.subcore_barrier()

        def chunk(j, carry):
            off = e0 + j * _K
            pltpu.sync_copy(src_hbm.at[pl.ds(off, _K)], sbuf)
            pltpu.sync_copy(dst_hbm.at[pl.ds(off, _K)], dbuf)
            pltpu.async_copy(x_hbm.at[sbuf], rows, gsem).wait()
            pltpu.sync_copy(rows, accum.at[dbuf], add=True)
            return carry

        lax.fori_loop(0, _CPW, chunk, 0)
        plsc.subcore_barrier()

        for j in range(_RPT // _ZR):
            r = row0 + j * _ZR
            pltpu.sync_copy(accum.at[pl.ds(r, _ZR)], rows)
            pltpu.sync_copy(rows, out_hbm.at[c].at[pl.ds(r, _ZR)])

    return k(x, src_p, dst_p)


def _mm_relu(partials, W):
    blk = 1024

    def body(p0_ref, p1_ref, w_ref, o_ref):
        ssum = p0_ref[...] + p1_ref[...]
        o_ref[...] = jnp.maximum(
            jnp.dot(ssum, w_ref[...], preferred_element_type=jnp.float32),
            0.0)

    return pl.pallas_call(
        body,
        grid=(_N_PAD // blk,),
        in_specs=[
            pl.BlockSpec((blk, _DIM), lambda i: (i, 0)),
            pl.BlockSpec((blk, _DIM), lambda i: (i, 0)),
            pl.BlockSpec((_DIM, _DIM), lambda i: (0, 0)),
        ],
        out_specs=pl.BlockSpec((blk, _DIM), lambda i: (i, 0)),
        out_shape=jax.ShapeDtypeStruct((_N_NODES, _DIM), jnp.float32),
    )(partials[0], partials[1], W)


def kernel(x, edge_index, W):
    src = edge_index[1].astype(jnp.int32)
    dst = edge_index[0].astype(jnp.int32)
    partials = _sc_aggregate(x, src, dst)
    return _mm_relu(partials, W)


# speedup vs baseline: 2.4211x; 1.2749x over previous
import functools

import jax
import jax.numpy as jnp
from jax import lax
from jax.experimental import pallas as pl
from jax.experimental.pallas import tpu as pltpu
from jax.experimental.pallas import tpu_sc as plsc

_N_NODES = 10000
_N_PAD = 10240
_N_EDGES = 320000
_DIM = 128
_NC = 2
_NS = 16
_NW = _NC * _NS
_K = 80
_CPW = 125
_RPT = _N_PAD // _NS
_ZR = _K


def _sc_aggregate(x, src_p, dst_p):
    mesh = plsc.VectorSubcoreMesh(core_axis_name="c", subcore_axis_name="s")

    @functools.partial(
        pl.kernel,
        out_type=jax.ShapeDtypeStruct((_NC, _N_PAD, _DIM), jnp.float32),
        mesh=mesh,
        scratch_types=[
            pltpu.VMEM_SHARED((_N_PAD, _DIM), jnp.float32),
            pltpu.VMEM((_K, _DIM), jnp.float32),
            [pltpu.VMEM((_K,), jnp.int32)] * 2,
            [pltpu.VMEM((_K,), jnp.int32)] * 2,
            pltpu.SemaphoreType.DMA,
            [pltpu.SemaphoreType.DMA] * 2,
            [pltpu.SemaphoreType.DMA] * 2,
        ],
    )
    def k(x_hbm, src_hbm, dst_hbm, out_hbm, accum, rows, sbuf, dbuf, gsem,
          ssem, dsem):
        c = lax.axis_index("c")
        s = lax.axis_index("s")
        w = s * _NC + c
        e0 = w * _CPW * _K

        def zero_row(r, carry):
            for j in range(_DIM // 16):
                rows[r, pl.ds(j * 16, 16)] = jnp.zeros((16,), jnp.float32)
            return carry

        lax.fori_loop(0, _ZR, zero_row, 0)
        row0 = s * _RPT
        for j in range(_RPT // _ZR):
            pltpu.sync_copy(rows, accum.at[pl.ds(row0 + j * _ZR, _ZR)])
        plsc.subcore_barrier()

        def sload(j, p):
            return pltpu.async_copy(
                src_hbm.at[pl.ds(e0 + j * _K, _K)], sbuf[p], ssem[p])

        def dload(j, p):
            return pltpu.async_copy(
                dst_hbm.at[pl.ds(e0 + j * _K, _K)], dbuf[p], dsem[p])

        def pair(i, carry):
            j0 = 2 * i
            ds0 = sload(j0, 0)
            dd0 = dload(j0, 0)
            ds0.wait()
            pltpu.async_copy(x_hbm.at[sbuf[0]], rows, gsem).wait()
            dd0.wait()
            ds1 = sload(j0 + 1, 1)
            dd1 = dload(j0 + 1, 1)
            pltpu.sync_copy(rows, accum.at[dbuf[0]], add=True)
            ds1.wait()
            pltpu.async_copy(x_hbm.at[sbuf[1]], rows, gsem).wait()
            dd1.wait()
            pltpu.sync_copy(rows, accum.at[dbuf[1]], add=True)
            return carry

        lax.fori_loop(0, _CPW // 2, pair, 0)
        sload(_CPW - 1, 0).wait()
        dd = dload(_CPW - 1, 0)
        pltpu.async_copy(x_hbm.at[sbuf[0]], rows, gsem).wait()
        dd.wait()
        pltpu.sync_copy(rows, accum.at[dbuf[0]], add=True)
        plsc.subcore_barrier()

        for j in range(_RPT // _ZR):
            r = row0 + j * _ZR
            pltpu.sync_copy(accum.at[pl.ds(r, _ZR)], rows)
            pltpu.sync_copy(rows, out_hbm.at[c].at[pl.ds(r, _ZR)])

    return k(x, src_p, dst_p)


def _mm_relu(partials, W):
    blk = 1024

    def body(p0_ref, p1_ref, w_ref, o_ref):
        ssum = p0_ref[...] + p1_ref[...]
        o_ref[...] = jnp.maximum(
            jnp.dot(ssum, w_ref[...], preferred_element_type=jnp.float32),
            0.0)

    return pl.pallas_call(
        body,
        grid=(_N_PAD // blk,),
        in_specs=[
            pl.BlockSpec((blk, _DIM), lambda i: (i, 0)),
            pl.BlockSpec((blk, _DIM), lambda i: (i, 0)),
            pl.BlockSpec((_DIM, _DIM), lambda i: (0, 0)),
        ],
        out_specs=pl.BlockSpec((blk, _DIM), lambda i: (i, 0)),
        out_shape=jax.ShapeDtypeStruct((_N_NODES, _DIM), jnp.float32),
    )(partials[0], partials[1], W)


def kernel(x, edge_index, W):
    src = edge_index[1].astype(jnp.int32)
    dst = edge_index[0].astype(jnp.int32)
    partials = _sc_aggregate(x, src, dst)
    return _mm_relu(partials, W)


# speedup vs baseline: 2.9627x; 1.2237x over previous
import functools

import jax
import jax.numpy as jnp
from jax import lax
from jax.experimental import pallas as pl
from jax.experimental.pallas import tpu as pltpu
from jax.experimental.pallas import tpu_sc as plsc

_N_NODES = 10000
_N_PAD = 10240
_N_EDGES = 320000
_DIM = 128
_NC = 2
_NS = 16
_NW = _NC * _NS
_K = 80
_CPW = 125
_RPT = _N_PAD // _NS
_ZR = _K
_NU = 4


def _sc_aggregate(x, src_p, dst_p):
    mesh = plsc.VectorSubcoreMesh(core_axis_name="c", subcore_axis_name="s")

    @functools.partial(
        pl.kernel,
        out_type=jax.ShapeDtypeStruct((_NC, _N_PAD, _DIM), jnp.float32),
        mesh=mesh,
        scratch_types=[
            pltpu.VMEM_SHARED((_N_PAD, _DIM), jnp.float32),
            [pltpu.VMEM((_K, _DIM), jnp.float32)] * 2,
            [pltpu.VMEM((_K,), jnp.int32)] * _NU,
            [pltpu.VMEM((_K,), jnp.int32)] * _NU,
            pltpu.SemaphoreType.DMA,
            pltpu.SemaphoreType.DMA,
            [pltpu.SemaphoreType.DMA] * _NU,
            [pltpu.SemaphoreType.DMA] * _NU,
        ],
    )
    def k(x_hbm, src_hbm, dst_hbm, out_hbm, accum, rows, sbuf, dbuf, gsem,
          scsem, ssem, dsem):
        c = lax.axis_index("c")
        s = lax.axis_index("s")
        w = s * _NC + c
        e0 = w * _CPW * _K

        def zero_row(r, carry):
            for j in range(_DIM // 16):
                rows[0][r, pl.ds(j * 16, 16)] = jnp.zeros((16,), jnp.float32)
            return carry

        lax.fori_loop(0, _ZR, zero_row, 0)
        row0 = s * _RPT
        for j in range(_RPT // _ZR):
            pltpu.sync_copy(rows[0], accum.at[pl.ds(row0 + j * _ZR, _ZR)])
        plsc.subcore_barrier()

        def sload(j, p):
            return pltpu.async_copy(
                src_hbm.at[pl.ds(e0 + j * _K, _K)], sbuf[p], ssem[p])

        def dload(j, p):
            return pltpu.async_copy(
                dst_hbm.at[pl.ds(e0 + j * _K, _K)], dbuf[p], dsem[p])

        def quad(i, carry):
            j0 = _NU * i
            sd = [sload(j0 + t, t) for t in range(_NU)]
            dd = [dload(j0 + t, t) for t in range(_NU)]
            sc = None
            for t in range(_NU):
                sd[t].wait()
                g = pltpu.async_copy(x_hbm.at[sbuf[t]], rows[t % 2], gsem)
                if sc is not None:
                    sc.wait()
                g.wait()
                dd[t].wait()
                sc = pltpu.async_copy(rows[t % 2], accum.at[dbuf[t]],
                                      scsem, add=True)
            sc.wait()
            return carry

        lax.fori_loop(0, _CPW // _NU, quad, 0)
        sload(_CPW - 1, 0).wait()
        dd = dload(_CPW - 1, 0)
        pltpu.async_copy(x_hbm.at[sbuf[0]], rows[0], gsem).wait()
        dd.wait()
        pltpu.sync_copy(rows[0], accum.at[dbuf[0]], add=True)
        plsc.subcore_barrier()

        for j in range(_RPT // _ZR):
            r = row0 + j * _ZR
            pltpu.sync_copy(accum.at[pl.ds(r, _ZR)], rows[0])
            pltpu.sync_copy(rows[0], out_hbm.at[c].at[pl.ds(r, _ZR)])

    return k(x, src_p, dst_p)


def _mm_relu(partials, W):
    blk = 1024

    def body(p0_ref, p1_ref, w_ref, o_ref):
        ssum = p0_ref[...] + p1_ref[...]
        o_ref[...] = jnp.maximum(
            jnp.dot(ssum, w_ref[...], preferred_element_type=jnp.float32),
            0.0)

    return pl.pallas_call(
        body,
        grid=(_N_PAD // blk,),
        in_specs=[
            pl.BlockSpec((blk, _DIM), lambda i: (i, 0)),
            pl.BlockSpec((blk, _DIM), lambda i: (i, 0)),
            pl.BlockSpec((_DIM, _DIM), lambda i: (0, 0)),
        ],
        out_specs=pl.BlockSpec((blk, _DIM), lambda i: (i, 0)),
        out_shape=jax.ShapeDtypeStruct((_N_NODES, _DIM), jnp.float32),
    )(partials[0], partials[1], W)


def kernel(x, edge_index, W):
    src = edge_index[1].astype(jnp.int32)
    dst = edge_index[0].astype(jnp.int32)
    partials = _sc_aggregate(x, src, dst)
    return _mm_relu(partials, W)


# speedup vs baseline: 3.0396x; 1.0260x over previous
import functools

import jax
import jax.numpy as jnp
from jax import lax
from jax.experimental import pallas as pl
from jax.experimental.pallas import tpu as pltpu
from jax.experimental.pallas import tpu_sc as plsc

_N_NODES = 10000
_N_PAD = 10240
_N_EDGES = 320000
_DIM = 128
_NC = 2
_NS = 16
_NW = _NC * _NS
_K = 80
_CPW = 125
_RPT = _N_PAD // _NS
_ZR = _K
_NU = 5


def _sc_aggregate(x, src_p, dst_p):
    mesh = plsc.VectorSubcoreMesh(core_axis_name="c", subcore_axis_name="s")

    @functools.partial(
        pl.kernel,
        out_type=jax.ShapeDtypeStruct((_NC, _N_PAD, _DIM), jnp.float32),
        mesh=mesh,
        scratch_types=[
            pltpu.VMEM_SHARED((_N_PAD, _DIM), jnp.float32),
            [pltpu.VMEM((_K, _DIM), jnp.float32)] * 2,
            [pltpu.VMEM((_K,), jnp.int32)] * _NU,
            [pltpu.VMEM((_K,), jnp.int32)] * _NU,
            pltpu.SemaphoreType.DMA,
            pltpu.SemaphoreType.DMA,
            [pltpu.SemaphoreType.DMA] * _NU,
            [pltpu.SemaphoreType.DMA] * _NU,
        ],
    )
    def k(x_hbm, src_hbm, dst_hbm, out_hbm, accum, rows, sbuf, dbuf, gsem,
          scsem, ssem, dsem):
        c = lax.axis_index("c")
        s = lax.axis_index("s")
        w = s * _NC + c
        e0 = w * _CPW * _K

        def zero_row(r, carry):
            for j in range(_DIM // 16):
                rows[0][r, pl.ds(j * 16, 16)] = jnp.zeros((16,), jnp.float32)
            return carry

        lax.fori_loop(0, _ZR, zero_row, 0)
        row0 = s * _RPT
        for j in range(_RPT // _ZR):
            pltpu.sync_copy(rows[0], accum.at[pl.ds(row0 + j * _ZR, _ZR)])
        plsc.subcore_barrier()

        def sload(j, p):
            return pltpu.async_copy(
                src_hbm.at[pl.ds(e0 + j * _K, _K)], sbuf[p], ssem[p])

        def dload(j, p):
            return pltpu.async_copy(
                dst_hbm.at[pl.ds(e0 + j * _K, _K)], dbuf[p], dsem[p])

        def quad(i, carry):
            j0 = _NU * i
            sd = [sload(j0 + t, t) for t in range(_NU)]
            dd = [dload(j0 + t, t) for t in range(_NU)]
            sc = None
            for t in range(_NU):
                sd[t].wait()
                g = pltpu.async_copy(x_hbm.at[sbuf[t]], rows[t % 2], gsem)
                if sc is not None:
                    sc.wait()
                g.wait()
                dd[t].wait()
                sc = pltpu.async_copy(rows[t % 2], accum.at[dbuf[t]],
                                      scsem, add=True)
            sc.wait()
            return carry

        lax.fori_loop(0, _CPW // _NU, quad, 0)
        plsc.subcore_barrier()

        for j in range(_RPT // _ZR):
            r = row0 + j * _ZR
            pltpu.sync_copy(accum.at[pl.ds(r, _ZR)], rows[0])
            pltpu.sync_copy(rows[0], out_hbm.at[c].at[pl.ds(r, _ZR)])

    return k(x, src_p, dst_p)


def _mm_relu(partials, W):
    blk = 1024

    def body(p0_ref, p1_ref, w_ref, o_ref):
        ssum = p0_ref[...] + p1_ref[...]
        o_ref[...] = jnp.maximum(
            jnp.dot(ssum, w_ref[...], preferred_element_type=jnp.float32),
            0.0)

    return pl.pallas_call(
        body,
        grid=(_N_PAD // blk,),
        in_specs=[
            pl.BlockSpec((blk, _DIM), lambda i: (i, 0)),
            pl.BlockSpec((blk, _DIM), lambda i: (i, 0)),
            pl.BlockSpec((_DIM, _DIM), lambda i: (0, 0)),
        ],
        out_specs=pl.BlockSpec((blk, _DIM), lambda i: (i, 0)),
        out_shape=jax.ShapeDtypeStruct((_N_NODES, _DIM), jnp.float32),
    )(partials[0], partials[1], W)


def kernel(x, edge_index, W):
    src = edge_index[1].astype(jnp.int32)
    dst = edge_index[0].astype(jnp.int32)
    partials = _sc_aggregate(x, src, dst)
    return _mm_relu(partials, W)


# speedup vs baseline: 3.0514x; 1.0039x over previous
import functools

import jax
import jax.numpy as jnp
from jax import lax
from jax.experimental import pallas as pl
from jax.experimental.pallas import tpu as pltpu
from jax.experimental.pallas import tpu_sc as plsc

_N_NODES = 10000
_N_PAD = 10240
_N_EDGES = 320000
_DIM = 128
_NC = 2
_NS = 16
_NW = _NC * _NS
_K = 80
_CPW = 125
_RPT = _N_PAD // _NS
_ZR = _K
_NU = 5


def _sc_aggregate(x, src_p, dst_p):
    mesh = plsc.VectorSubcoreMesh(core_axis_name="c", subcore_axis_name="s")

    @functools.partial(
        pl.kernel,
        out_type=jax.ShapeDtypeStruct((_NC, _N_PAD, _DIM), jnp.float32),
        mesh=mesh,
        scratch_types=[
            pltpu.VMEM_SHARED((_N_PAD, _DIM), jnp.float32),
            [pltpu.VMEM((_K, _DIM), jnp.float32)] * 2,
            [pltpu.VMEM((_K,), jnp.int32)] * _NU,
            [pltpu.VMEM((_K,), jnp.int32)] * _NU,
            pltpu.SemaphoreType.DMA,
            pltpu.SemaphoreType.DMA,
            [pltpu.SemaphoreType.DMA] * _NU,
            [pltpu.SemaphoreType.DMA] * _NU,
        ],
    )
    def k(x_hbm, src_hbm, dst_hbm, out_hbm, accum, rows, sbuf, dbuf, gsem,
          scsem, ssem, dsem):
        c = lax.axis_index("c")
        s = lax.axis_index("s")
        w = s * _NC + c
        e0 = w * _CPW * _K

        def zero_row(r, carry):
            for j in range(_DIM // 16):
                rows[0][r, pl.ds(j * 16, 16)] = jnp.zeros((16,), jnp.float32)
            return carry

        lax.fori_loop(0, _ZR, zero_row, 0)
        row0 = s * _RPT
        zd = [
            pltpu.async_copy(
                rows[0], accum.at[pl.ds(row0 + j * _ZR, _ZR)], gsem)
            for j in range(_RPT // _ZR)
        ]
        for d in zd:
            d.wait()
        plsc.subcore_barrier()

        def sload(j, p):
            return pltpu.async_copy(
                src_hbm.at[pl.ds(e0 + j * _K, _K)], sbuf[p], ssem[p])

        def dload(j, p):
            return pltpu.async_copy(
                dst_hbm.at[pl.ds(e0 + j * _K, _K)], dbuf[p], dsem[p])

        def quad(i, carry):
            j0 = _NU * i
            sd = [sload(j0 + t, t) for t in range(_NU)]
            dd = [dload(j0 + t, t) for t in range(_NU)]
            sc = None
            for t in range(_NU):
                sd[t].wait()
                g = pltpu.async_copy(x_hbm.at[sbuf[t]], rows[t % 2], gsem)
                if sc is not None:
                    sc.wait()
                g.wait()
                dd[t].wait()
                sc = pltpu.async_copy(rows[t % 2], accum.at[dbuf[t]],
                                      scsem, add=True)
            sc.wait()
            return carry

        lax.fori_loop(0, _CPW // _NU, quad, 0)
        plsc.subcore_barrier()

        wd = [
            pltpu.async_copy(
                accum.at[pl.ds(row0 + j * _ZR, _ZR)],
                out_hbm.at[c].at[pl.ds(row0 + j * _ZR, _ZR)], gsem)
            for j in range(_RPT // _ZR)
        ]
        for d in wd:
            d.wait()

    return k(x, src_p, dst_p)


def _mm_relu(partials, W):
    blk = 1024

    def body(p0_ref, p1_ref, w_ref, o_ref):
        ssum = p0_ref[...] + p1_ref[...]
        o_ref[...] = jnp.maximum(
            jnp.dot(ssum, w_ref[...], preferred_element_type=jnp.float32),
            0.0)

    return pl.pallas_call(
        body,
        grid=(_N_PAD // blk,),
        in_specs=[
            pl.BlockSpec((blk, _DIM), lambda i: (i, 0)),
            pl.BlockSpec((blk, _DIM), lambda i: (i, 0)),
            pl.BlockSpec((_DIM, _DIM), lambda i: (0, 0)),
        ],
        out_specs=pl.BlockSpec((blk, _DIM), lambda i: (i, 0)),
        out_shape=jax.ShapeDtypeStruct((_N_NODES, _DIM), jnp.float32),
    )(partials[0], partials[1], W)


def kernel(x, edge_index, W):
    src = edge_index[1].astype(jnp.int32)
    dst = edge_index[0].astype(jnp.int32)
    partials = _sc_aggregate(x, src, dst)
    return _mm_relu(partials, W)


# speedup vs baseline: 3.0863x; 1.0114x over previous
import functools

import jax
import jax.numpy as jnp
from jax import lax
from jax.experimental import pallas as pl
from jax.experimental.pallas import tpu as pltpu
from jax.experimental.pallas import tpu_sc as plsc

_N_NODES = 10000
_N_PAD = 10240
_N_EDGES = 320000
_DIM = 128
_NC = 2
_NS = 16
_NW = _NC * _NS
_K = 80
_CPW = 125
_RPT = _N_PAD // _NS
_ZR = _K
_NU = 5


def _sc_aggregate(x, src_p, dst_p):
    mesh = plsc.VectorSubcoreMesh(core_axis_name="c", subcore_axis_name="s")

    @functools.partial(
        pl.kernel,
        out_type=jax.ShapeDtypeStruct((_NC, _N_PAD, _DIM), jnp.float32),
        mesh=mesh,
        scratch_types=[
            pltpu.VMEM_SHARED((_N_PAD, _DIM), jnp.float32),
            [pltpu.VMEM((_K, _DIM), jnp.float32)] * 2,
            [pltpu.VMEM((_K,), jnp.int32)] * _NU,
            [pltpu.VMEM((_K,), jnp.int32)] * _NU,
            pltpu.SemaphoreType.DMA,
            pltpu.SemaphoreType.DMA,
            [pltpu.SemaphoreType.DMA] * _NU,
            [pltpu.SemaphoreType.DMA] * _NU,
        ],
    )
    def k(x_hbm, src_hbm, dst_hbm, out_hbm, accum, rows, sbuf, dbuf, gsem,
          scsem, ssem, dsem):
        c = lax.axis_index("c")
        s = lax.axis_index("s")
        w = s * _NC + c
        e0 = w * _CPW * _K

        def zero_row(r, carry):
            for j in range(_DIM // 16):
                rows[0][r, pl.ds(j * 16, 16)] = jnp.zeros((16,), jnp.float32)
            return carry

        lax.fori_loop(0, _ZR, zero_row, 0)
        row0 = s * _RPT
        zd = [
            pltpu.async_copy(
                rows[0], accum.at[pl.ds(row0 + j * _ZR, _ZR)], gsem)
            for j in range(_RPT // _ZR)
        ]
        for d in zd:
            d.wait()
        plsc.subcore_barrier()

        def sload(j, p):
            return pltpu.async_copy(
                src_hbm.at[pl.ds(e0 + j * _K, _K)], sbuf[p], ssem[p])

        def dload(j, p):
            return pltpu.async_copy(
                dst_hbm.at[pl.ds(e0 + j * _K, _K)], dbuf[p], dsem[p])

        def quad(i, carry):
            j0 = _NU * i
            sd = [sload(j0 + t, t) for t in range(_NU)]
            dd = [dload(j0 + t, t) for t in range(_NU)]
            sc = None
            for t in range(_NU):
                sd[t].wait()
                g = pltpu.async_copy(x_hbm.at[sbuf[t]], rows[t % 2], gsem)
                if sc is not None:
                    sc.wait()
                g.wait()
                dd[t].wait()
                sc = pltpu.async_copy(rows[t % 2], accum.at[dbuf[t]],
                                      scsem, add=True)
            sc.wait()
            return carry

        lax.fori_loop(0, _CPW // _NU, quad, 0)
        plsc.subcore_barrier()

        wd = [
            pltpu.async_copy(
                accum.at[pl.ds(row0 + j * _ZR, _ZR)],
                out_hbm.at[c].at[pl.ds(row0 + j * _ZR, _ZR)], gsem)
            for j in range(_RPT // _ZR)
        ]
        for d in wd:
            d.wait()

    return k(x, src_p, dst_p)


def _mm_relu(partials, W):
    blk = 2048

    def body(p0_ref, p1_ref, w_ref, o_ref):
        ssum = p0_ref[...] + p1_ref[...]
        o_ref[...] = jnp.maximum(
            jnp.dot(ssum, w_ref[...], preferred_element_type=jnp.float32),
            0.0)

    return pl.pallas_call(
        body,
        grid=(_N_PAD // blk,),
        in_specs=[
            pl.BlockSpec((blk, _DIM), lambda i: (i, 0)),
            pl.BlockSpec((blk, _DIM), lambda i: (i, 0)),
            pl.BlockSpec((_DIM, _DIM), lambda i: (0, 0)),
        ],
        out_specs=pl.BlockSpec((blk, _DIM), lambda i: (i, 0)),
        out_shape=jax.ShapeDtypeStruct((_N_NODES, _DIM), jnp.float32),
    )(partials[0], partials[1], W)


def kernel(x, edge_index, W):
    src = edge_index[1].astype(jnp.int32)
    dst = edge_index[0].astype(jnp.int32)
    partials = _sc_aggregate(x, src, dst)
    return _mm_relu(partials, W)
